# baseline (speedup 1.0000x reference)
import functools

import jax
import jax.numpy as jnp
from jax import lax
from jax.experimental import pallas as pl
from jax.experimental.pallas import tpu as pltpu
from jax.experimental.pallas import tpu_sc as plsc

L = 16


def _vsqrt(x):
    i = plsc.bitcast(x, jnp.int32)
    i = jnp.int32(0x5F3759DF) - (i >> 1)
    y = plsc.bitcast(i, jnp.float32)
    hx = 0.5 * x
    for _ in range(3):
        y = y * (1.5 - hx * y * y)
    return x * y


def kernel(heads, relations, tails, entity_emb, relation_emb):
    B = heads.shape[0]
    D = entity_emb.shape[1]
    NREL = relation_emb.shape[0]
    info = plsc.get_sparse_core_info()
    NC, NS = info.num_cores, info.num_subcores
    NW = NC * NS
    BPW = B // NW
    CH = 128
    SIZES = (32, 96, 128, 128, 96, 32)
    STARTS = (0, 32, 128, 256, 384, 480)
    NCHUNK = len(SIZES)
    assert sum(SIZES) == BPW and all(s % L == 0 for s in SIZES)
    assert B % (8 * NW) == 0

    mesh = plsc.VectorSubcoreMesh(core_axis_name="c", subcore_axis_name="s")

    @functools.partial(
        pl.kernel,
        out_type=jax.ShapeDtypeStruct((B,), jnp.float32),
        mesh=mesh,
        compiler_params=pltpu.CompilerParams(needs_layout_passes=False),
        scratch_types=[
            pltpu.VMEM((3 * BPW,), jnp.int32),
            pltpu.VMEM((3, CH, D), jnp.float32),
            pltpu.VMEM((3, CH, D), jnp.float32),
            pltpu.VMEM((BPW,), jnp.float32),
            pltpu.VMEM_SHARED((NREL, D), jnp.float32),
            pltpu.SemaphoreType.DMA,
            pltpu.SemaphoreType.DMA,
            pltpu.SemaphoreType.DMA,
            pltpu.SemaphoreType.DMA,
            pltpu.SemaphoreType.DMA,
            pltpu.SemaphoreType.DMA,
        ],
    )
    def run(hrt_h, ent_h, rel_h, out_h,
            idxs, hbuf, tbuf, out_v, rel_s,
            sem_h0, sem_h1, sem_h2, sem_t0, sem_t1, sem_t2):
        wid = lax.axis_index("s") * NC + lax.axis_index("c")
        base = wid * BPW
        pltpu.sync_copy(hrt_h.at[pl.ds(wid * (3 * BPW), 3 * BPW)], idxs)

        sems_h = (sem_h0, sem_h1, sem_h2)
        sems_t = (sem_t0, sem_t1, sem_t2)
        NB = 3
        iota = lax.iota(jnp.int32, L)

        def fire_ht(c):
            b, st, sz = c % NB, STARTS[c], SIZES[c]
            pltpu.async_copy(ent_h.at[idxs.at[pl.ds(st, sz)]],
                             hbuf.at[b].at[pl.ds(0, sz)], sems_h[b])
            pltpu.async_copy(ent_h.at[idxs.at[pl.ds(2 * BPW + st, sz)]],
                             tbuf.at[b].at[pl.ds(0, sz)], sems_t[b])

        def fire_radd(c):
            b, st, sz = c % NB, STARTS[c], SIZES[c]
            pltpu.make_async_copy(
                ent_h.at[pl.ds(0, sz)], hbuf.at[b].at[pl.ds(0, sz)], sems_h[b]
            ).wait()
            pltpu.async_copy(
                rel_s.at[idxs.at[pl.ds(BPW + st, sz)]],
                hbuf.at[b].at[pl.ds(0, sz)], sems_t[b], add=True
            )

        def drain_tr(c):
            b, sz = c % NB, SIZES[c]
            pltpu.make_async_copy(
                ent_h.at[pl.ds(0, sz)], tbuf.at[b].at[pl.ds(0, sz)], sems_t[b]
            ).wait()
            pltpu.make_async_copy(
                rel_s.at[pl.ds(0, sz)], hbuf.at[b].at[pl.ds(0, sz)], sems_t[b]
            ).wait()

        def compute(c, b, half):
            hb, tb = hbuf.at[b], tbuf.at[b]

            def gbody(g, _):
                rowv = iota + g * L

                def kbody(_, carry):
                    acc, dbase = carry
                    rot = iota
                    for _s in range(L):
                        dv = dbase + rot
                        vhr = plsc.load_gather(hb, [rowv, dv])
                        vt = plsc.load_gather(tb, [rowv, dv])
                        diff = vhr - vt
                        acc = acc + diff * diff
                        rot = (rot + 1) & (L - 1)
                    return acc, dbase + L

                acc, _ = lax.fori_loop(
                    0, D // L, kbody,
                    (jnp.zeros((L,), jnp.float32), jnp.zeros((L,), jnp.int32)),
                )
                out_v[pl.ds(STARTS[c] + g * L, L)] = _vsqrt(acc)
                return 0

            ng = SIZES[c] // L
            lax.fori_loop(half * (ng // 2), (half + 1) * (ng // 2), gbody, 0)

        fire_ht(0)
        if NCHUNK > 1:
            fire_ht(1)

        @pl.when(lax.axis_index("s") == 0)
        def _():
            pltpu.sync_copy(rel_h, rel_s)

        plsc.subcore_barrier()

        fire_radd(0)
        for c in range(NCHUNK):
            drain_tr(c)
            compute(c, c % NB, 0)
            if c + 1 < NCHUNK:
                fire_radd(c + 1)
            compute(c, c % NB, 1)
            if c + 2 < NCHUNK:
                fire_ht(c + 2)

        pltpu.sync_copy(out_v, out_h.at[pl.ds(base, BPW)])

    hrt = jnp.stack([heads.astype(jnp.int32).reshape(NW, BPW),
                     relations.astype(jnp.int32).reshape(NW, BPW),
                     tails.astype(jnp.int32).reshape(NW, BPW)],
                    axis=1).reshape(-1)
    return run(hrt, entity_emb, relation_emb)


# speedup vs baseline: 1.0143x; 1.0143x over previous
import functools

import jax
import jax.numpy as jnp
from jax import lax
from jax.experimental import pallas as pl
from jax.experimental.pallas import tpu as pltpu
from jax.experimental.pallas import tpu_sc as plsc

L = 16


def _vsqrt(x):
    i = plsc.bitcast(x, jnp.int32)
    i = jnp.int32(0x5F3759DF) - (i >> 1)
    y = plsc.bitcast(i, jnp.float32)
    hx = 0.5 * x
    for _ in range(3):
        y = y * (1.5 - hx * y * y)
    return x * y


def kernel(heads, relations, tails, entity_emb, relation_emb):
    B = heads.shape[0]
    D = entity_emb.shape[1]
    NREL = relation_emb.shape[0]
    info = plsc.get_sparse_core_info()
    NC, NS = info.num_cores, info.num_subcores
    NW = NC * NS
    BPW = B // NW
    CH = 128
    SIZES = (64, 128, 128, 128, 64)
    STARTS = (0, 64, 192, 320, 448)
    NCHUNK = len(SIZES)
    assert sum(SIZES) == BPW and all(s % L == 0 for s in SIZES)
    assert B % (8 * NW) == 0

    mesh = plsc.VectorSubcoreMesh(core_axis_name="c", subcore_axis_name="s")

    @functools.partial(
        pl.kernel,
        out_type=jax.ShapeDtypeStruct((B,), jnp.float32),
        mesh=mesh,
        compiler_params=pltpu.CompilerParams(needs_layout_passes=False),
        scratch_types=[
            pltpu.VMEM((3 * BPW,), jnp.int32),
            pltpu.VMEM((3, CH, D), jnp.float32),
            pltpu.VMEM((3, CH, D), jnp.float32),
            pltpu.VMEM((BPW,), jnp.float32),
            pltpu.VMEM_SHARED((NREL, D), jnp.float32),
            pltpu.SemaphoreType.DMA,
            pltpu.SemaphoreType.DMA,
            pltpu.SemaphoreType.DMA,
            pltpu.SemaphoreType.DMA,
            pltpu.SemaphoreType.DMA,
            pltpu.SemaphoreType.DMA,
        ],
    )
    def run(hrt_h, ent_h, rel_h, out_h,
            idxs, hbuf, tbuf, out_v, rel_s,
            sem_h0, sem_h1, sem_h2, sem_t0, sem_t1, sem_t2):
        wid = lax.axis_index("s") * NC + lax.axis_index("c")
        base = wid * BPW
        pltpu.sync_copy(hrt_h.at[pl.ds(wid * (3 * BPW), 3 * BPW)], idxs)

        sems_h = (sem_h0, sem_h1, sem_h2)
        sems_t = (sem_t0, sem_t1, sem_t2)
        NB = 3
        iota = lax.iota(jnp.int32, L)

        def fire_ht(c):
            b, st, sz = c % NB, STARTS[c], SIZES[c]
            pltpu.async_copy(ent_h.at[idxs.at[pl.ds(st, sz)]],
                             hbuf.at[b].at[pl.ds(0, sz)], sems_h[b])
            pltpu.async_copy(ent_h.at[idxs.at[pl.ds(2 * BPW + st, sz)]],
                             tbuf.at[b].at[pl.ds(0, sz)], sems_t[b])

        def fire_radd(c):
            b, st, sz = c % NB, STARTS[c], SIZES[c]
            pltpu.make_async_copy(
                ent_h.at[pl.ds(0, sz)], hbuf.at[b].at[pl.ds(0, sz)], sems_h[b]
            ).wait()
            pltpu.async_copy(
                rel_s.at[idxs.at[pl.ds(BPW + st, sz)]],
                hbuf.at[b].at[pl.ds(0, sz)], sems_t[b], add=True
            )

        def drain_tr(c):
            b, sz = c % NB, SIZES[c]
            pltpu.make_async_copy(
                ent_h.at[pl.ds(0, sz)], tbuf.at[b].at[pl.ds(0, sz)], sems_t[b]
            ).wait()
            pltpu.make_async_copy(
                rel_s.at[pl.ds(0, sz)], hbuf.at[b].at[pl.ds(0, sz)], sems_t[b]
            ).wait()

        def compute(c, b, half):
            hb, tb = hbuf.at[b], tbuf.at[b]

            def gbody(g, _):
                rowv = iota + g * L

                def kbody(_, carry):
                    acc, dbase = carry
                    rot = iota
                    for _s in range(L):
                        dv = dbase + rot
                        vhr = plsc.load_gather(hb, [rowv, dv])
                        vt = plsc.load_gather(tb, [rowv, dv])
                        diff = vhr - vt
                        acc = acc + diff * diff
                        rot = (rot + 1) & (L - 1)
                    return acc, dbase + L

                acc, _ = lax.fori_loop(
                    0, D // L, kbody,
                    (jnp.zeros((L,), jnp.float32), jnp.zeros((L,), jnp.int32)),
                )
                out_v[pl.ds(STARTS[c] + g * L, L)] = _vsqrt(acc)
                return 0

            ng = SIZES[c] // L
            lax.fori_loop(half * (ng // 2), (half + 1) * (ng // 2), gbody, 0)

        fire_ht(0)
        if NCHUNK > 1:
            fire_ht(1)

        @pl.when(lax.axis_index("s") == 0)
        def _():
            pltpu.sync_copy(rel_h, rel_s)

        plsc.subcore_barrier()

        fire_radd(0)
        for c in range(NCHUNK):
            drain_tr(c)
            compute(c, c % NB, 0)
            if c + 1 < NCHUNK:
                fire_radd(c + 1)
            compute(c, c % NB, 1)
            if c + 2 < NCHUNK:
                fire_ht(c + 2)

        pltpu.sync_copy(out_v, out_h.at[pl.ds(base, BPW)])

    hrt = jnp.stack([heads.astype(jnp.int32).reshape(NW, BPW),
                     relations.astype(jnp.int32).reshape(NW, BPW),
                     tails.astype(jnp.int32).reshape(NW, BPW)],
                    axis=1).reshape(-1)
    return run(hrt, entity_emb, relation_emb)
